# baseline (speedup 1.0000x reference)
import jax
import jax.numpy as jnp
from jax import lax
from jax.experimental import pallas as pl
from jax.experimental.pallas import tpu as pltpu
from jax.experimental.pallas import tpu_sc as plsc

_NC = 2
_NS = 16
_NW = _NC * _NS
_L = 16

_VOCAB = 160
_VPAD = 256
_D = 1024
_B = 4 * 8192
_C = 64

_B_SC = 8192
_BPW = _B_SC // _NW
_NCHUNK = _BPW // _C

_B_TC = _B - _B_SC
_TBLK = 1024

_NMASK = _B // _NW // _C


def _sc_body(idx_hbm, midx_hbm, table_hbm, out_hbm, mask_hbm,
             idx_v, mask_v, rows_v, gsem, osem):
    wid = lax.axis_index("s") * _NC + lax.axis_index("c")
    base = wid * _BPW

    pltpu.sync_copy(idx_hbm.at[wid], idx_v)

    def chunk(c, carry):
        pltpu.async_copy(table_hbm.at[idx_v.at[c]], rows_v, gsem)
        pltpu.async_copy(rows_v, out_hbm.at[pl.ds(base + c * _C, _C)], osem)
        return carry

    lax.fori_loop(0, _NCHUNK, chunk, 0)

    pltpu.sync_copy(midx_hbm.at[wid], mask_v)

    def mrow(c, carry):
        for j in range(_C // _L):
            v = mask_v[c, pl.ds(j * _L, _L)]
            mask_v[c, pl.ds(j * _L, _L)] = jnp.where(
                v == 0, jnp.int32(1), jnp.int32(0))
        return carry

    lax.fori_loop(0, _NMASK, mrow, 0)
    pltpu.sync_copy(mask_v, mask_hbm.at[wid])

    def drain(c, carry):
        pltpu.make_async_copy(table_hbm.at[idx_v.at[0]], rows_v, gsem).wait()
        pltpu.make_async_copy(rows_v, out_hbm.at[pl.ds(base, _C)],
                              osem).wait()
        return carry

    lax.fori_loop(0, _NCHUNK, drain, 0)


def _tc_body(toks_ref, hi_ref, lo_ref, out_ref):
    toks = toks_ref[0, 0, :]
    iota = lax.broadcasted_iota(jnp.int32, (1, _VPAD), 1)
    onehot = jnp.equal(toks[:, None], iota).astype(
        jnp.float32).astype(jnp.bfloat16)
    out_ref[...] = (
        jnp.dot(onehot, hi_ref[...], preferred_element_type=jnp.float32)
        + jnp.dot(onehot, lo_ref[...], preferred_element_type=jnp.float32))


def kernel(token_id, embed_weight):
    tid = token_id.astype(jnp.int32)
    flat = tid.reshape(_B)
    idx_sc = flat[:_B_SC].reshape(_NW, _NCHUNK, _C)
    idx_all = flat.reshape(_NW, _NMASK, _C)
    toks_tc = flat[_B_SC:].reshape(_B_TC // _TBLK, 1, _TBLK)

    mesh = plsc.VectorSubcoreMesh(core_axis_name="c", subcore_axis_name="s")
    sc_out, mask = pl.kernel(
        _sc_body,
        out_type=[
            jax.ShapeDtypeStruct((_B_SC, _D), jnp.float32),
            jax.ShapeDtypeStruct((_NW, _NMASK, _C), jnp.int32),
        ],
        mesh=mesh,
        scratch_types=[
            pltpu.VMEM((_NCHUNK, _C), jnp.int32),
            pltpu.VMEM((_NMASK, _C), jnp.int32),
            pltpu.VMEM((_C, _D), jnp.float32),
            pltpu.SemaphoreType.DMA,
            pltpu.SemaphoreType.DMA,
        ],
    )(idx_sc, idx_all, embed_weight)

    wpad = jnp.pad(embed_weight, ((0, _VPAD - _VOCAB), (0, 0)))
    table_hi = wpad.astype(jnp.bfloat16)
    table_lo = (wpad - table_hi.astype(jnp.float32)).astype(jnp.bfloat16)
    tc_out = pl.pallas_call(
        _tc_body,
        grid=(_B_TC // _TBLK,),
        in_specs=[
            pl.BlockSpec((1, 1, _TBLK), lambda i: (i, 0, 0)),
            pl.BlockSpec((_VPAD, _D), lambda i: (0, 0)),
            pl.BlockSpec((_VPAD, _D), lambda i: (0, 0)),
        ],
        out_specs=pl.BlockSpec((_TBLK, _D),
                               lambda i: (i + _B_SC // _TBLK, 0)),
        out_shape=jax.ShapeDtypeStruct((_B, _D), jnp.float32),
    )(toks_tc, table_hi, table_lo)

    x = lax.dynamic_update_slice(tc_out, sc_out, (0, 0))
    x = x.reshape(token_id.shape[0], token_id.shape[1], _D)
    padding_mask = mask.reshape(token_id.shape).astype(bool)
    return (x, padding_mask, token_id)


# speedup vs baseline: 1.0664x; 1.0664x over previous
import jax
import jax.numpy as jnp
from jax import lax
from jax.experimental import pallas as pl
from jax.experimental.pallas import tpu as pltpu
from jax.experimental.pallas import tpu_sc as plsc

_NC = 2
_NS = 16
_NW = _NC * _NS
_L = 16

_VOCAB = 160
_VPAD = 256
_D = 1024
_B = 4 * 8192
_C = 64

_B_SC = 8192
_BPW = _B_SC // _NW
_NCHUNK = _BPW // _C

_B_TC = _B - _B_SC
_TBLK = 2048

_NMASK = _B // _NW // _C


def _sc_body(idx_hbm, midx_hbm, table_hbm, out_hbm, mask_hbm,
             idx_v, mask_v, rows_v, gsem, osem):
    wid = lax.axis_index("s") * _NC + lax.axis_index("c")
    base = wid * _BPW

    pltpu.sync_copy(idx_hbm.at[wid], idx_v)

    def chunk(c, carry):
        pltpu.async_copy(table_hbm.at[idx_v.at[c]], rows_v, gsem)
        pltpu.async_copy(rows_v, out_hbm.at[pl.ds(base + c * _C, _C)], osem)
        return carry

    lax.fori_loop(0, _NCHUNK, chunk, 0)

    pltpu.sync_copy(midx_hbm.at[wid], mask_v)

    def mrow(c, carry):
        for j in range(_C // _L):
            v = mask_v[c, pl.ds(j * _L, _L)]
            mask_v[c, pl.ds(j * _L, _L)] = jnp.where(
                v == 0, jnp.int32(1), jnp.int32(0))
        return carry

    lax.fori_loop(0, _NMASK, mrow, 0)
    pltpu.sync_copy(mask_v, mask_hbm.at[wid])

    def drain(c, carry):
        pltpu.make_async_copy(table_hbm.at[idx_v.at[0]], rows_v, gsem).wait()
        pltpu.make_async_copy(rows_v, out_hbm.at[pl.ds(base, _C)],
                              osem).wait()
        return carry

    lax.fori_loop(0, _NCHUNK, drain, 0)


def _tc_body(toks_ref, table_ref, out_ref):
    toks = toks_ref[0, 0, :]
    iota = lax.broadcasted_iota(jnp.int32, (1, _VOCAB), 1)
    onehot = jnp.equal(toks[:, None], iota).astype(jnp.float32)
    out_ref[...] = jnp.dot(onehot, table_ref[...],
                           preferred_element_type=jnp.float32)


def kernel(token_id, embed_weight):
    tid = token_id.astype(jnp.int32)
    flat = tid.reshape(_B)
    idx_sc = flat[:_B_SC].reshape(_NW, _NCHUNK, _C)
    idx_all = flat.reshape(_NW, _NMASK, _C)
    toks_tc = flat[_B_SC:].reshape(_B_TC // _TBLK, 1, _TBLK)

    mesh = plsc.VectorSubcoreMesh(core_axis_name="c", subcore_axis_name="s")
    sc_out, mask = pl.kernel(
        _sc_body,
        out_type=[
            jax.ShapeDtypeStruct((_B_SC, _D), jnp.float32),
            jax.ShapeDtypeStruct((_NW, _NMASK, _C), jnp.int32),
        ],
        mesh=mesh,
        scratch_types=[
            pltpu.VMEM((_NCHUNK, _C), jnp.int32),
            pltpu.VMEM((_NMASK, _C), jnp.int32),
            pltpu.VMEM((_C, _D), jnp.float32),
            pltpu.SemaphoreType.DMA,
            pltpu.SemaphoreType.DMA,
        ],
    )(idx_sc, idx_all, embed_weight)

    tc_out = pl.pallas_call(
        _tc_body,
        grid=(_B_TC // _TBLK,),
        in_specs=[
            pl.BlockSpec((1, 1, _TBLK), lambda i: (i, 0, 0)),
            pl.BlockSpec((_VOCAB, _D), lambda i: (0, 0)),
        ],
        out_specs=pl.BlockSpec((_TBLK, _D),
                               lambda i: (i + _B_SC // _TBLK, 0)),
        out_shape=jax.ShapeDtypeStruct((_B, _D), jnp.float32),
    )(toks_tc, embed_weight)

    x = lax.dynamic_update_slice(tc_out, sc_out, (0, 0))
    x = x.reshape(token_id.shape[0], token_id.shape[1], _D)
    padding_mask = mask.reshape(token_id.shape).astype(bool)
    return (x, padding_mask, token_id)


# speedup vs baseline: 1.1798x; 1.1063x over previous
import jax
import jax.numpy as jnp
from jax import lax
from jax.experimental import pallas as pl
from jax.experimental.pallas import tpu as pltpu
from jax.experimental.pallas import tpu_sc as plsc

_NC = 2
_NS = 16
_NW = _NC * _NS
_L = 16

_VOCAB = 160
_VPAD = 256
_D = 1024
_B = 4 * 8192
_C = 64

_B_SC = 6144
_BPW = _B_SC // _NW
_NCHUNK = _BPW // _C

_B_TC = _B - _B_SC
_TBLK = 2048

_NMASK = _B // _NW // _C


def _sc_body(idx_hbm, midx_hbm, table_hbm, out_hbm, mask_hbm,
             idx_v, mask_v, rows_v, gsem, osem):
    wid = lax.axis_index("s") * _NC + lax.axis_index("c")
    base = wid * _BPW

    pltpu.sync_copy(idx_hbm.at[wid], idx_v)

    def chunk(c, carry):
        pltpu.async_copy(table_hbm.at[idx_v.at[c]], rows_v, gsem)
        pltpu.async_copy(rows_v, out_hbm.at[pl.ds(base + c * _C, _C)], osem)
        return carry

    lax.fori_loop(0, _NCHUNK, chunk, 0)

    pltpu.sync_copy(midx_hbm.at[wid], mask_v)

    def mrow(c, carry):
        for j in range(_C // _L):
            v = mask_v[c, pl.ds(j * _L, _L)]
            mask_v[c, pl.ds(j * _L, _L)] = jnp.where(
                v == 0, jnp.int32(1), jnp.int32(0))
        return carry

    lax.fori_loop(0, _NMASK, mrow, 0)
    pltpu.sync_copy(mask_v, mask_hbm.at[wid])

    def drain(c, carry):
        pltpu.make_async_copy(table_hbm.at[idx_v.at[0]], rows_v, gsem).wait()
        pltpu.make_async_copy(rows_v, out_hbm.at[pl.ds(base, _C)],
                              osem).wait()
        return carry

    lax.fori_loop(0, _NCHUNK, drain, 0)


def _tc_body(toks_ref, table_ref, out_ref):
    toks = toks_ref[0, 0, :]
    iota = lax.broadcasted_iota(jnp.int32, (1, _VOCAB), 1)
    onehot = jnp.equal(toks[:, None], iota).astype(jnp.float32)
    out_ref[...] = jnp.dot(onehot, table_ref[...],
                           preferred_element_type=jnp.float32)


def kernel(token_id, embed_weight):
    tid = token_id.astype(jnp.int32)
    flat = tid.reshape(_B)
    idx_sc = flat[:_B_SC].reshape(_NW, _NCHUNK, _C)
    idx_all = flat.reshape(_NW, _NMASK, _C)
    toks_tc = flat[_B_SC:].reshape(_B_TC // _TBLK, 1, _TBLK)

    mesh = plsc.VectorSubcoreMesh(core_axis_name="c", subcore_axis_name="s")
    sc_out, mask = pl.kernel(
        _sc_body,
        out_type=[
            jax.ShapeDtypeStruct((_B_SC, _D), jnp.float32),
            jax.ShapeDtypeStruct((_NW, _NMASK, _C), jnp.int32),
        ],
        mesh=mesh,
        scratch_types=[
            pltpu.VMEM((_NCHUNK, _C), jnp.int32),
            pltpu.VMEM((_NMASK, _C), jnp.int32),
            pltpu.VMEM((_C, _D), jnp.float32),
            pltpu.SemaphoreType.DMA,
            pltpu.SemaphoreType.DMA,
        ],
    )(idx_sc, idx_all, embed_weight)

    tc_out = pl.pallas_call(
        _tc_body,
        grid=(_B_TC // _TBLK,),
        in_specs=[
            pl.BlockSpec((1, 1, _TBLK), lambda i: (i, 0, 0)),
            pl.BlockSpec((_VOCAB, _D), lambda i: (0, 0)),
        ],
        out_specs=pl.BlockSpec((_TBLK, _D),
                               lambda i: (i + _B_SC // _TBLK, 0)),
        out_shape=jax.ShapeDtypeStruct((_B, _D), jnp.float32),
    )(toks_tc, embed_weight)

    x = lax.dynamic_update_slice(tc_out, sc_out, (0, 0))
    x = x.reshape(token_id.shape[0], token_id.shape[1], _D)
    padding_mask = mask.reshape(token_id.shape).astype(bool)
    return (x, padding_mask, token_id)


# speedup vs baseline: 1.3173x; 1.1166x over previous
import jax
import jax.numpy as jnp
from jax import lax
from jax.experimental import pallas as pl
from jax.experimental.pallas import tpu as pltpu
from jax.experimental.pallas import tpu_sc as plsc

_NC = 2
_NS = 16
_NW = _NC * _NS
_L = 16

_VOCAB = 160
_VPAD = 256
_D = 1024
_B = 4 * 8192
_C = 64

_B_SC = 4096
_BPW = _B_SC // _NW
_NCHUNK = _BPW // _C

_B_TC = _B - _B_SC
_TBLK = 2048

_NMASK = _B // _NW // _C


def _sc_body(idx_hbm, midx_hbm, table_hbm, out_hbm, mask_hbm,
             idx_v, mask_v, rows_v, gsem, osem):
    wid = lax.axis_index("s") * _NC + lax.axis_index("c")
    base = wid * _BPW

    pltpu.sync_copy(idx_hbm.at[wid], idx_v)

    def chunk(c, carry):
        pltpu.async_copy(table_hbm.at[idx_v.at[c]], rows_v, gsem)
        pltpu.async_copy(rows_v, out_hbm.at[pl.ds(base + c * _C, _C)], osem)
        return carry

    lax.fori_loop(0, _NCHUNK, chunk, 0)

    pltpu.sync_copy(midx_hbm.at[wid], mask_v)

    def mrow(c, carry):
        for j in range(_C // _L):
            v = mask_v[c, pl.ds(j * _L, _L)]
            mask_v[c, pl.ds(j * _L, _L)] = jnp.where(
                v == 0, jnp.int32(1), jnp.int32(0))
        return carry

    lax.fori_loop(0, _NMASK, mrow, 0)
    pltpu.sync_copy(mask_v, mask_hbm.at[wid])

    def drain(c, carry):
        pltpu.make_async_copy(table_hbm.at[idx_v.at[0]], rows_v, gsem).wait()
        pltpu.make_async_copy(rows_v, out_hbm.at[pl.ds(base, _C)],
                              osem).wait()
        return carry

    lax.fori_loop(0, _NCHUNK, drain, 0)


def _tc_body(toks_ref, table_ref, out_ref):
    toks = toks_ref[0, 0, :]
    iota = lax.broadcasted_iota(jnp.int32, (1, _VOCAB), 1)
    onehot = jnp.equal(toks[:, None], iota).astype(jnp.float32)
    out_ref[...] = jnp.dot(onehot, table_ref[...],
                           preferred_element_type=jnp.float32)


def kernel(token_id, embed_weight):
    tid = token_id.astype(jnp.int32)
    flat = tid.reshape(_B)
    idx_sc = flat[:_B_SC].reshape(_NW, _NCHUNK, _C)
    idx_all = flat.reshape(_NW, _NMASK, _C)
    toks_tc = flat[_B_SC:].reshape(_B_TC // _TBLK, 1, _TBLK)

    mesh = plsc.VectorSubcoreMesh(core_axis_name="c", subcore_axis_name="s")
    sc_out, mask = pl.kernel(
        _sc_body,
        out_type=[
            jax.ShapeDtypeStruct((_B_SC, _D), jnp.float32),
            jax.ShapeDtypeStruct((_NW, _NMASK, _C), jnp.int32),
        ],
        mesh=mesh,
        scratch_types=[
            pltpu.VMEM((_NCHUNK, _C), jnp.int32),
            pltpu.VMEM((_NMASK, _C), jnp.int32),
            pltpu.VMEM((_C, _D), jnp.float32),
            pltpu.SemaphoreType.DMA,
            pltpu.SemaphoreType.DMA,
        ],
    )(idx_sc, idx_all, embed_weight)

    tc_out = pl.pallas_call(
        _tc_body,
        grid=(_B_TC // _TBLK,),
        in_specs=[
            pl.BlockSpec((1, 1, _TBLK), lambda i: (i, 0, 0)),
            pl.BlockSpec((_VOCAB, _D), lambda i: (0, 0)),
        ],
        out_specs=pl.BlockSpec((_TBLK, _D),
                               lambda i: (i + _B_SC // _TBLK, 0)),
        out_shape=jax.ShapeDtypeStruct((_B, _D), jnp.float32),
    )(toks_tc, embed_weight)

    x = lax.dynamic_update_slice(tc_out, sc_out, (0, 0))
    x = x.reshape(token_id.shape[0], token_id.shape[1], _D)
    padding_mask = mask.reshape(token_id.shape).astype(bool)
    return (x, padding_mask, token_id)
